# baseline (speedup 1.0000x reference)
import functools

import jax
import jax.numpy as jnp
from jax import lax
from jax.experimental import pallas as pl
from jax.experimental.pallas import tpu as pltpu
from jax.experimental.pallas import tpu_sc as plsc

N = 10000
D = 128
E = 320000
GW = 144
NT = 32
NSUB = 16
CH = 128
_KCH0 = (E + NT * CH - 1) // (NT * CH)
KCH = ((_KCH0 + 1) // 2) * 2
EP = NT * CH * KCH
NPAD = N + 112
SLICE = NPAD // NSUB
BLK = 1000


def _prep_body(x_ref, wlin_ref, blin_ref, war_ref, scal_ref, h_ref, g_ref):
    x = x_ref[...]
    h = lax.dot_general(x, wlin_ref[...], (((1,), (1,)), ((), ())),
                        preferred_element_type=jnp.float32) + blin_ref[...]
    h_ref[...] = h
    ar = jnp.sum(h * war_ref[...], axis=1, keepdims=True) + scal_ref[0]
    u = jnp.exp(scal_ref[1] * ar)
    g_ref[...] = jnp.concatenate(
        [u * h, jnp.broadcast_to(u, (BLK, GW - D))], axis=1)


def _prep(x, W_lin, b_lin, W_attn_r, scal):
    return pl.pallas_call(
        _prep_body,
        grid=(N // BLK,),
        in_specs=[
            pl.BlockSpec((BLK, D), lambda i: (i, 0)),
            pl.BlockSpec((D, D), lambda i: (0, 0)),
            pl.BlockSpec((1, D), lambda i: (0, 0)),
            pl.BlockSpec((1, D), lambda i: (0, 0)),
            pl.BlockSpec(memory_space=pltpu.SMEM),
        ],
        out_specs=[
            pl.BlockSpec((BLK, D), lambda i: (i, 0)),
            pl.BlockSpec((BLK, GW), lambda i: (i, 0)),
        ],
        out_shape=[
            jax.ShapeDtypeStruct((N, D), jnp.float32),
            jax.ShapeDtypeStruct((N, GW), jnp.float32),
        ],
    )(x, W_lin, b_lin, W_attn_r, scal)


NB = 2
NBLK = KCH // NB


def _sc_body(g_hbm, src_hbm, dst_hbm, zeros_hbm, out_hbm,
             src_v, dst_v, rows_v, acc_sh, *sems):
    c = lax.axis_index("c")
    s = lax.axis_index("s")
    wid = c * NSUB + s

    row0 = pl.multiple_of(s * SLICE, 8)
    pltpu.sync_copy(zeros_hbm, acc_sh.at[pl.ds(row0, SLICE)])

    pltpu.sync_copy(src_hbm.at[wid], src_v)
    pltpu.sync_copy(dst_hbm.at[wid], dst_v)

    def body(kk, carry):
        for b in range(NB):
            pltpu.sync_copy(g_hbm.at[src_v.at[kk, b]], rows_v)
            pltpu.sync_copy(rows_v, acc_sh.at[dst_v.at[kk, b]], add=True)
        return carry

    lax.fori_loop(0, NBLK, body, 0)

    plsc.subcore_barrier()
    pltpu.sync_copy(acc_sh.at[pl.ds(row0, SLICE)],
                    out_hbm.at[c, pl.ds(row0, SLICE)])


def _sc_aggregate(g, src_p, dst_p, zeros):
    mesh = plsc.VectorSubcoreMesh(core_axis_name="c", subcore_axis_name="s")
    kern = pl.kernel(
        _sc_body,
        out_type=jax.ShapeDtypeStruct((2, NPAD, GW), jnp.float32),
        mesh=mesh,
        scratch_types=[
            pltpu.VMEM((NBLK, NB, CH), jnp.int32),
            pltpu.VMEM((NBLK, NB, CH), jnp.int32),
            pltpu.VMEM((CH, GW), jnp.float32),
            pltpu.VMEM_SHARED((NPAD, GW), jnp.float32),
        ],
        compiler_params=pltpu.CompilerParams(use_tc_tiling_on_sc=False),
    )
    return kern(g, src_p, dst_p, zeros)


def _combine_body(s_ref, h_ref, x_ref, wc_ref, bc_ref, o_ref):
    ssum = s_ref[0] + s_ref[1]
    agg = ssum[:, :D] / jnp.maximum(ssum[:, D:D + 1], 1e-16)
    wd = wc_ref[0:1, :] - wc_ref[1:2, :]
    dlt = jnp.sum(x_ref[...] * wd, axis=1, keepdims=True) + (bc_ref[0] - bc_ref[1])
    beta0 = 1.0 / (1.0 + jnp.exp(-dlt))
    out = beta0 * agg + (1.0 - beta0) * h_ref[...]
    o_ref[...] = jnp.maximum(out, 0.0)


def _combine(S, h, x, W_conv_pad, b_conv):
    return pl.pallas_call(
        _combine_body,
        grid=(N // BLK,),
        in_specs=[
            pl.BlockSpec((2, BLK, GW), lambda i: (0, i, 0)),
            pl.BlockSpec((BLK, D), lambda i: (i, 0)),
            pl.BlockSpec((BLK, D), lambda i: (i, 0)),
            pl.BlockSpec((8, D), lambda i: (0, 0)),
            pl.BlockSpec(memory_space=pltpu.SMEM),
        ],
        out_specs=pl.BlockSpec((BLK, D), lambda i: (i, 0)),
        out_shape=jax.ShapeDtypeStruct((N, D), jnp.float32),
    )(S, h, x, W_conv_pad, b_conv)


def kernel(x, global_node_idx, edge_index, W_lin, b_lin, W_conv, b_conv,
           W_attn_l, b_attn_l, W_attn_r, b_attn_r, alpha_weights):
    scal = jnp.stack([b_attn_r.astype(jnp.float32).reshape(()),
                      alpha_weights.astype(jnp.float32).reshape(())])
    h, g = _prep(x, W_lin, b_lin.reshape(1, D), W_attn_r.reshape(1, D), scal)

    dst = edge_index[0]
    src = edge_index[1]
    pad = EP - E
    src_p = jnp.concatenate([src, jnp.zeros((pad,), jnp.int32)]).reshape(NT, NBLK, NB, CH)
    dst_p = jnp.concatenate([dst, jnp.full((pad,), N, jnp.int32)]).reshape(NT, NBLK, NB, CH)
    zeros = jnp.zeros((SLICE, GW), jnp.float32)

    S = _sc_aggregate(g, src_p, dst_p, zeros)

    W_conv_pad = jnp.zeros((8, D), jnp.float32).at[:2].set(W_conv)
    return _combine(S, h, x, W_conv_pad, b_conv)


# speedup vs baseline: 2.4277x; 2.4277x over previous
import functools

import jax
import jax.numpy as jnp
from jax import lax
from jax.experimental import pallas as pl
from jax.experimental.pallas import tpu as pltpu
from jax.experimental.pallas import tpu_sc as plsc

N = 10000
D = 128
E = 320000
GW = 144
NT = 32
NSUB = 16
CH = 128
EPT = E // NT
KCH = EPT // CH
TAIL = EPT - KCH * CH
NPAD = N + 112
SLICE = NPAD // NSUB
BLK = 1000


def _prep_body(x_ref, wlin_ref, blin_ref, war_ref, scal_ref, h_ref, g_ref):
    x = x_ref[...]
    h = lax.dot_general(x, wlin_ref[...], (((1,), (1,)), ((), ())),
                        preferred_element_type=jnp.float32) + blin_ref[...]
    h_ref[...] = h
    ar = jnp.sum(h * war_ref[...], axis=1, keepdims=True) + scal_ref[0]
    u = jnp.exp(scal_ref[1] * ar)
    g_ref[...] = jnp.concatenate(
        [u * h, jnp.broadcast_to(u, (BLK, GW - D))], axis=1)


def _prep(x, W_lin, b_lin, W_attn_r, scal):
    return pl.pallas_call(
        _prep_body,
        grid=(N // BLK,),
        in_specs=[
            pl.BlockSpec((BLK, D), lambda i: (i, 0)),
            pl.BlockSpec((D, D), lambda i: (0, 0)),
            pl.BlockSpec((1, D), lambda i: (0, 0)),
            pl.BlockSpec((1, D), lambda i: (0, 0)),
            pl.BlockSpec(memory_space=pltpu.SMEM),
        ],
        out_specs=[
            pl.BlockSpec((BLK, D), lambda i: (i, 0)),
            pl.BlockSpec((BLK, GW), lambda i: (i, 0)),
        ],
        out_shape=[
            jax.ShapeDtypeStruct((N, D), jnp.float32),
            jax.ShapeDtypeStruct((N, GW), jnp.float32),
        ],
    )(x, W_lin, b_lin, W_attn_r, scal)


def _sc_body(g_hbm, src_hbm, dst_hbm, tsrc_hbm, tdst_hbm, zeros_hbm, out_hbm,
             src_v, dst_v, tsrc_v, tdst_v, rows_v, acc_sh):
    c = lax.axis_index("c")
    s = lax.axis_index("s")
    wid = c * NSUB + s

    row0 = pl.multiple_of(s * SLICE, 8)
    pltpu.sync_copy(zeros_hbm, acc_sh.at[pl.ds(row0, SLICE)])

    pltpu.sync_copy(src_hbm.at[wid], src_v)
    pltpu.sync_copy(dst_hbm.at[wid], dst_v)
    pltpu.sync_copy(tsrc_hbm.at[wid], tsrc_v)
    pltpu.sync_copy(tdst_hbm.at[wid], tdst_v)

    def body(k, carry):
        pltpu.sync_copy(g_hbm.at[src_v.at[k]], rows_v)
        pltpu.sync_copy(rows_v, acc_sh.at[dst_v.at[k]], add=True)
        return carry

    lax.fori_loop(0, KCH, body, 0)

    pltpu.sync_copy(g_hbm.at[tsrc_v], rows_v.at[pl.ds(0, TAIL)])
    pltpu.sync_copy(rows_v.at[pl.ds(0, TAIL)], acc_sh.at[tdst_v], add=True)

    plsc.subcore_barrier()
    pltpu.sync_copy(acc_sh.at[pl.ds(row0, SLICE)],
                    out_hbm.at[c, pl.ds(row0, SLICE)])


def _sc_aggregate(g, src_m, dst_m, src_t, dst_t, zeros):
    mesh = plsc.VectorSubcoreMesh(core_axis_name="c", subcore_axis_name="s")
    kern = pl.kernel(
        _sc_body,
        out_type=jax.ShapeDtypeStruct((2, NPAD, GW), jnp.float32),
        mesh=mesh,
        scratch_types=[
            pltpu.VMEM((KCH, CH), jnp.int32),
            pltpu.VMEM((KCH, CH), jnp.int32),
            pltpu.VMEM((TAIL,), jnp.int32),
            pltpu.VMEM((TAIL,), jnp.int32),
            pltpu.VMEM((CH, GW), jnp.float32),
            pltpu.VMEM_SHARED((NPAD, GW), jnp.float32),
        ],
        compiler_params=pltpu.CompilerParams(use_tc_tiling_on_sc=False),
    )
    return kern(g, src_m, dst_m, src_t, dst_t, zeros)


def _combine_body(s_ref, h_ref, x_ref, wc_ref, bc_ref, o_ref):
    ssum = s_ref[0] + s_ref[1]
    agg = ssum[:, :D] / jnp.maximum(ssum[:, D:D + 1], 1e-16)
    wd = wc_ref[0:1, :] - wc_ref[1:2, :]
    dlt = jnp.sum(x_ref[...] * wd, axis=1, keepdims=True) + (bc_ref[0] - bc_ref[1])
    beta0 = 1.0 / (1.0 + jnp.exp(-dlt))
    out = beta0 * agg + (1.0 - beta0) * h_ref[...]
    o_ref[...] = jnp.maximum(out, 0.0)


def _combine(S, h, x, W_conv_pad, b_conv):
    return pl.pallas_call(
        _combine_body,
        grid=(N // BLK,),
        in_specs=[
            pl.BlockSpec((2, BLK, GW), lambda i: (0, i, 0)),
            pl.BlockSpec((BLK, D), lambda i: (i, 0)),
            pl.BlockSpec((BLK, D), lambda i: (i, 0)),
            pl.BlockSpec((8, D), lambda i: (0, 0)),
            pl.BlockSpec(memory_space=pltpu.SMEM),
        ],
        out_specs=pl.BlockSpec((BLK, D), lambda i: (i, 0)),
        out_shape=jax.ShapeDtypeStruct((N, D), jnp.float32),
    )(S, h, x, W_conv_pad, b_conv)


def kernel(x, global_node_idx, edge_index, W_lin, b_lin, W_conv, b_conv,
           W_attn_l, b_attn_l, W_attn_r, b_attn_r, alpha_weights):
    scal = jnp.stack([b_attn_r.astype(jnp.float32).reshape(()),
                      alpha_weights.astype(jnp.float32).reshape(())])
    h, g = _prep(x, W_lin, b_lin.reshape(1, D), W_attn_r.reshape(1, D), scal)

    dst = edge_index[0].reshape(NT, EPT)
    src = edge_index[1].reshape(NT, EPT)
    src_m = src[:, :KCH * CH].reshape(NT, KCH, CH)
    dst_m = dst[:, :KCH * CH].reshape(NT, KCH, CH)
    src_t = src[:, KCH * CH:]
    dst_t = dst[:, KCH * CH:]
    zeros = jnp.zeros((SLICE, GW), jnp.float32)

    S = _sc_aggregate(g, src_m, dst_m, src_t, dst_t, zeros)

    W_conv_pad = jnp.zeros((8, D), jnp.float32).at[:2].set(W_conv)
    return _combine(S, h, x, W_conv_pad, b_conv)


# speedup vs baseline: 2.5933x; 1.0682x over previous
import functools

import jax
import jax.numpy as jnp
from jax import lax
from jax.experimental import pallas as pl
from jax.experimental.pallas import tpu as pltpu
from jax.experimental.pallas import tpu_sc as plsc

N = 10000
D = 128
E = 320000
GW = 144
NT = 32
NSUB = 16
CH = 128
EPT = E // NT
KCH = EPT // CH
TAIL = EPT - KCH * CH
NPAD = N + 112
SLICE = NPAD // NSUB
BLK = 1000


def _prep_body(x_ref, wlin_ref, blin_ref, war_ref, scal_ref, h_ref, g_ref):
    x = x_ref[...]
    h = lax.dot_general(x, wlin_ref[...], (((1,), (1,)), ((), ())),
                        preferred_element_type=jnp.float32) + blin_ref[...]
    h_ref[...] = h
    ar = jnp.sum(h * war_ref[...], axis=1, keepdims=True) + scal_ref[0]
    u = jnp.exp(scal_ref[1] * ar)
    g_ref[...] = jnp.concatenate(
        [u * h, jnp.broadcast_to(u, (BLK, GW - D))], axis=1)


def _prep(x, W_lin, b_lin, W_attn_r, scal):
    return pl.pallas_call(
        _prep_body,
        grid=(N // BLK,),
        in_specs=[
            pl.BlockSpec((BLK, D), lambda i: (i, 0)),
            pl.BlockSpec((D, D), lambda i: (0, 0)),
            pl.BlockSpec((1, D), lambda i: (0, 0)),
            pl.BlockSpec((1, D), lambda i: (0, 0)),
            pl.BlockSpec(memory_space=pltpu.SMEM),
        ],
        out_specs=[
            pl.BlockSpec((BLK, D), lambda i: (i, 0)),
            pl.BlockSpec((BLK, GW), lambda i: (i, 0)),
        ],
        out_shape=[
            jax.ShapeDtypeStruct((N, D), jnp.float32),
            jax.ShapeDtypeStruct((N, GW), jnp.float32),
        ],
    )(x, W_lin, b_lin, W_attn_r, scal)


def _sc_body(g_hbm, edge_hbm, zeros_hbm, out0_hbm, out1_hbm,
             src_v, dst_v, rows_v, acc_sh):
    c = lax.axis_index("c")
    s = lax.axis_index("s")
    wid = c * NSUB + s
    e0 = wid * EPT

    row0 = pl.multiple_of(s * SLICE, 8)
    pltpu.sync_copy(zeros_hbm, acc_sh.at[pl.ds(row0, SLICE)])

    pltpu.sync_copy(edge_hbm.at[1, pl.ds(e0, EPT)], src_v)
    pltpu.sync_copy(edge_hbm.at[0, pl.ds(e0, EPT)], dst_v)

    def body(k, carry):
        sl = pl.ds(k * CH, CH)
        pltpu.sync_copy(g_hbm.at[src_v.at[sl]], rows_v)
        pltpu.sync_copy(rows_v, acc_sh.at[dst_v.at[sl]], add=True)
        return carry

    lax.fori_loop(0, KCH, body, 0)

    tl = pl.ds(KCH * CH, TAIL)
    pltpu.sync_copy(g_hbm.at[src_v.at[tl]], rows_v.at[pl.ds(0, TAIL)])
    pltpu.sync_copy(rows_v.at[pl.ds(0, TAIL)], acc_sh.at[dst_v.at[tl]],
                    add=True)

    plsc.subcore_barrier()

    @pl.when(c == 0)
    def _():
        pltpu.sync_copy(acc_sh.at[pl.ds(row0, SLICE)],
                        out0_hbm.at[pl.ds(row0, SLICE)])

    @pl.when(c == 1)
    def _():
        pltpu.sync_copy(acc_sh.at[pl.ds(row0, SLICE)],
                        out1_hbm.at[pl.ds(row0, SLICE)])


def _sc_aggregate(g, edge_index, zeros):
    mesh = plsc.VectorSubcoreMesh(core_axis_name="c", subcore_axis_name="s")
    kern = pl.kernel(
        _sc_body,
        out_type=[jax.ShapeDtypeStruct((NPAD, GW), jnp.float32),
                  jax.ShapeDtypeStruct((NPAD, GW), jnp.float32)],
        mesh=mesh,
        scratch_types=[
            pltpu.VMEM((EPT,), jnp.int32),
            pltpu.VMEM((EPT,), jnp.int32),
            pltpu.VMEM((CH, GW), jnp.float32),
            pltpu.VMEM_SHARED((NPAD, GW), jnp.float32),
        ],
        compiler_params=pltpu.CompilerParams(use_tc_tiling_on_sc=False),
    )
    return kern(g, edge_index, zeros)


def _combine_body(s0_ref, s1_ref, h_ref, x_ref, wc_ref, bc_ref, o_ref):
    ssum = s0_ref[...] + s1_ref[...]
    agg = ssum[:, :D] / jnp.maximum(ssum[:, D:D + 1], 1e-16)
    wd = wc_ref[0:1, :] - wc_ref[1:2, :]
    dlt = jnp.sum(x_ref[...] * wd, axis=1, keepdims=True) + (bc_ref[0] - bc_ref[1])
    beta0 = 1.0 / (1.0 + jnp.exp(-dlt))
    out = beta0 * agg + (1.0 - beta0) * h_ref[...]
    o_ref[...] = jnp.maximum(out, 0.0)


def _combine(S0, S1, h, x, W_conv_pad, b_conv):
    return pl.pallas_call(
        _combine_body,
        grid=(N // BLK,),
        in_specs=[
            pl.BlockSpec((BLK, GW), lambda i: (i, 0)),
            pl.BlockSpec((BLK, GW), lambda i: (i, 0)),
            pl.BlockSpec((BLK, D), lambda i: (i, 0)),
            pl.BlockSpec((BLK, D), lambda i: (i, 0)),
            pl.BlockSpec((8, D), lambda i: (0, 0)),
            pl.BlockSpec(memory_space=pltpu.SMEM),
        ],
        out_specs=pl.BlockSpec((BLK, D), lambda i: (i, 0)),
        out_shape=jax.ShapeDtypeStruct((N, D), jnp.float32),
    )(S0, S1, h, x, W_conv_pad, b_conv)


def kernel(x, global_node_idx, edge_index, W_lin, b_lin, W_conv, b_conv,
           W_attn_l, b_attn_l, W_attn_r, b_attn_r, alpha_weights):
    scal = jnp.stack([b_attn_r.astype(jnp.float32).reshape(()),
                      alpha_weights.astype(jnp.float32).reshape(())])
    h, g = _prep(x, W_lin, b_lin.reshape(1, D), W_attn_r.reshape(1, D), scal)

    zeros = jnp.zeros((SLICE, GW), jnp.float32)
    S0, S1 = _sc_aggregate(g, edge_index, zeros)

    W_conv_pad = jnp.zeros((8, D), jnp.float32).at[:2].set(W_conv)
    return _combine(S0, S1, h, x, W_conv_pad, b_conv)


# speedup vs baseline: 3.2166x; 1.2403x over previous
import functools

import jax
import jax.numpy as jnp
from jax import lax
from jax.experimental import pallas as pl
from jax.experimental.pallas import tpu as pltpu
from jax.experimental.pallas import tpu_sc as plsc

N = 10000
D = 128
E = 320000
GW = 144
NT = 32
NSUB = 16
CH = 64
EPT = E // NT
KCH = EPT // CH
TAIL = EPT - KCH * CH
NPAD = N + 112
SLICE = NPAD // NSUB
BLK = 1000


def _prep_body(x_ref, wlin_ref, blin_ref, war_ref, scal_ref, h_ref, g_ref):
    x = x_ref[...]
    h = lax.dot_general(x, wlin_ref[...], (((1,), (1,)), ((), ())),
                        preferred_element_type=jnp.float32) + blin_ref[...]
    h_ref[...] = h
    ar = jnp.sum(h * war_ref[...], axis=1, keepdims=True) + scal_ref[0]
    u = jnp.exp(scal_ref[1] * ar)
    g_ref[...] = jnp.concatenate(
        [u * h, jnp.broadcast_to(u, (BLK, GW - D))], axis=1)


def _prep(x, W_lin, b_lin, W_attn_r, scal):
    return pl.pallas_call(
        _prep_body,
        grid=(N // BLK,),
        in_specs=[
            pl.BlockSpec((BLK, D), lambda i: (i, 0)),
            pl.BlockSpec((D, D), lambda i: (0, 0)),
            pl.BlockSpec((1, D), lambda i: (0, 0)),
            pl.BlockSpec((1, D), lambda i: (0, 0)),
            pl.BlockSpec(memory_space=pltpu.SMEM),
        ],
        out_specs=[
            pl.BlockSpec((BLK, D), lambda i: (i, 0)),
            pl.BlockSpec((BLK, GW), lambda i: (i, 0)),
        ],
        out_shape=[
            jax.ShapeDtypeStruct((N, D), jnp.float32),
            jax.ShapeDtypeStruct((N, GW), jnp.float32),
        ],
    )(x, W_lin, b_lin, W_attn_r, scal)


def _sc_body(g_hbm, edge_hbm, zeros_hbm, out0_hbm, out1_hbm,
             src_v, dst_v, rows_v, acc_sh, gs0, gs1, ss0, ss1):
    gsem = (gs0, gs1)
    ssem = (ss0, ss1)
    c = lax.axis_index("c")
    s = lax.axis_index("s")
    wid = c * NSUB + s
    e0 = wid * EPT

    row0 = pl.multiple_of(s * SLICE, 8)
    pltpu.sync_copy(zeros_hbm, acc_sh.at[pl.ds(row0, SLICE)])

    pltpu.sync_copy(edge_hbm.at[1, pl.ds(e0, EPT)], src_v)
    pltpu.sync_copy(edge_hbm.at[0, pl.ds(e0, EPT)], dst_v)

    def g_pair(k, b):
        return (g_hbm.at[src_v.at[pl.ds(k * CH, CH)]], rows_v.at[b], gsem[b])

    def s_pair(k, b):
        return (rows_v.at[b], acc_sh.at[dst_v.at[pl.ds(k * CH, CH)]],
                ssem[b])

    def g_issue(k, b):
        pltpu.async_copy(*g_pair(k, b))

    def g_wait(k, b):
        pltpu.make_async_copy(*g_pair(k, b)).wait()

    def s_issue(k, b):
        pltpu.async_copy(*s_pair(k, b), add=True)

    def s_wait(k, b):
        pltpu.make_async_copy(*s_pair(k, b)).wait()

    tl = pl.ds(KCH * CH, TAIL)

    def gt_pair():
        return (g_hbm.at[src_v.at[tl]], rows_v.at[0, pl.ds(0, TAIL)],
                gsem[0])

    def st_pair():
        return (rows_v.at[0, pl.ds(0, TAIL)], acc_sh.at[dst_v.at[tl]],
                ssem[0])

    g_issue(0, 0)
    g_wait(0, 0)
    s_issue(0, 0)
    g_issue(1, 1)

    def body(kk, carry):
        for j in range(2):
            k = 2 * kk + 1 + j
            cur = (1 + j) % 2
            oth = 1 - cur
            s_wait(k - 1, oth)
            g_issue(k + 1, oth)
            g_wait(k, cur)
            s_issue(k, cur)
        return carry

    lax.fori_loop(0, (KCH - 2) // 2, body, 0)

    s_wait(KCH - 2, 0)
    pltpu.async_copy(*gt_pair())
    g_wait(KCH - 1, 1)
    s_issue(KCH - 1, 1)
    pltpu.make_async_copy(*gt_pair()).wait()
    pltpu.async_copy(*st_pair(), add=True)
    s_wait(KCH - 1, 1)
    pltpu.make_async_copy(*st_pair()).wait()

    plsc.subcore_barrier()

    @pl.when(c == 0)
    def _():
        pltpu.sync_copy(acc_sh.at[pl.ds(row0, SLICE)],
                        out0_hbm.at[pl.ds(row0, SLICE)])

    @pl.when(c == 1)
    def _():
        pltpu.sync_copy(acc_sh.at[pl.ds(row0, SLICE)],
                        out1_hbm.at[pl.ds(row0, SLICE)])


def _sc_aggregate(g, edge_index, zeros):
    mesh = plsc.VectorSubcoreMesh(core_axis_name="c", subcore_axis_name="s")
    kern = pl.kernel(
        _sc_body,
        out_type=[jax.ShapeDtypeStruct((NPAD, GW), jnp.float32),
                  jax.ShapeDtypeStruct((NPAD, GW), jnp.float32)],
        mesh=mesh,
        scratch_types=[
            pltpu.VMEM((EPT,), jnp.int32),
            pltpu.VMEM((EPT,), jnp.int32),
            pltpu.VMEM((2, CH, GW), jnp.float32),
            pltpu.VMEM_SHARED((NPAD, GW), jnp.float32),
            *([pltpu.SemaphoreType.DMA] * 4),
        ],
        compiler_params=pltpu.CompilerParams(use_tc_tiling_on_sc=False),
    )
    return kern(g, edge_index, zeros)


def _combine_body(s0_ref, s1_ref, h_ref, x_ref, wc_ref, bc_ref, o_ref):
    ssum = s0_ref[...] + s1_ref[...]
    agg = ssum[:, :D] / jnp.maximum(ssum[:, D:D + 1], 1e-16)
    wd = wc_ref[0:1, :] - wc_ref[1:2, :]
    dlt = jnp.sum(x_ref[...] * wd, axis=1, keepdims=True) + (bc_ref[0] - bc_ref[1])
    beta0 = 1.0 / (1.0 + jnp.exp(-dlt))
    out = beta0 * agg + (1.0 - beta0) * h_ref[...]
    o_ref[...] = jnp.maximum(out, 0.0)


def _combine(S0, S1, h, x, W_conv_pad, b_conv):
    return pl.pallas_call(
        _combine_body,
        grid=(N // BLK,),
        in_specs=[
            pl.BlockSpec((BLK, GW), lambda i: (i, 0)),
            pl.BlockSpec((BLK, GW), lambda i: (i, 0)),
            pl.BlockSpec((BLK, D), lambda i: (i, 0)),
            pl.BlockSpec((BLK, D), lambda i: (i, 0)),
            pl.BlockSpec((8, D), lambda i: (0, 0)),
            pl.BlockSpec(memory_space=pltpu.SMEM),
        ],
        out_specs=pl.BlockSpec((BLK, D), lambda i: (i, 0)),
        out_shape=jax.ShapeDtypeStruct((N, D), jnp.float32),
    )(S0, S1, h, x, W_conv_pad, b_conv)


def kernel(x, global_node_idx, edge_index, W_lin, b_lin, W_conv, b_conv,
           W_attn_l, b_attn_l, W_attn_r, b_attn_r, alpha_weights):
    scal = jnp.stack([b_attn_r.astype(jnp.float32).reshape(()),
                      alpha_weights.astype(jnp.float32).reshape(())])
    h, g = _prep(x, W_lin, b_lin.reshape(1, D), W_attn_r.reshape(1, D), scal)

    zeros = jnp.zeros((SLICE, GW), jnp.float32)
    S0, S1 = _sc_aggregate(g, edge_index, zeros)

    W_conv_pad = jnp.zeros((8, D), jnp.float32).at[:2].set(W_conv)
    return _combine(S0, S1, h, x, W_conv_pad, b_conv)


# speedup vs baseline: 3.4581x; 1.0751x over previous
import functools

import jax
import jax.numpy as jnp
from jax import lax
from jax.experimental import pallas as pl
from jax.experimental.pallas import tpu as pltpu
from jax.experimental.pallas import tpu_sc as plsc

N = 10000
D = 128
E = 320000
DW = 16
NT = 32
NSUB = 16
CH = 64
EPT = E // NT
KCH = EPT // CH
TAIL = EPT - KCH * CH
NPAD = N + 112
SLICE = NPAD // NSUB
BLK = 1000


def _prep_body(x_ref, wlin_ref, blin_ref, war_ref, scal_ref,
               h_ref, g_ref, u_ref):
    x = x_ref[...]
    h = lax.dot_general(x, wlin_ref[...], (((1,), (1,)), ((), ())),
                        preferred_element_type=jnp.float32) + blin_ref[...]
    h_ref[...] = h
    ar = jnp.sum(h * war_ref[...], axis=1, keepdims=True) + scal_ref[0]
    u = jnp.exp(scal_ref[1] * ar)
    g_ref[...] = u * h
    u_ref[...] = jnp.broadcast_to(u, (BLK, DW))


def _prep(x, W_lin, b_lin, W_attn_r, scal):
    return pl.pallas_call(
        _prep_body,
        grid=(N // BLK,),
        in_specs=[
            pl.BlockSpec((BLK, D), lambda i: (i, 0)),
            pl.BlockSpec((D, D), lambda i: (0, 0)),
            pl.BlockSpec((1, D), lambda i: (0, 0)),
            pl.BlockSpec((1, D), lambda i: (0, 0)),
            pl.BlockSpec(memory_space=pltpu.SMEM),
        ],
        out_specs=[
            pl.BlockSpec((BLK, D), lambda i: (i, 0)),
            pl.BlockSpec((BLK, D), lambda i: (i, 0)),
            pl.BlockSpec((BLK, DW), lambda i: (i, 0)),
        ],
        out_shape=[
            jax.ShapeDtypeStruct((N, D), jnp.float32),
            jax.ShapeDtypeStruct((N, D), jnp.float32),
            jax.ShapeDtypeStruct((N, DW), jnp.float32),
        ],
    )(x, W_lin, b_lin, W_attn_r, scal)


def _sc_body(g_hbm, u_hbm, edge_hbm, zeros_hbm, zerosd_hbm,
             out0_hbm, out1_hbm, den0_hbm, den1_hbm,
             src_v, dst_v, rows_v, drows_v, acc_sh, accd_sh,
             gs0, gs1, ss0, ss1):
    gsem = (gs0, gs1)
    ssem = (ss0, ss1)
    c = lax.axis_index("c")
    s = lax.axis_index("s")
    wid = c * NSUB + s
    e0 = wid * EPT

    row0 = pl.multiple_of(s * SLICE, 8)
    pltpu.sync_copy(zeros_hbm, acc_sh.at[pl.ds(row0, SLICE)])
    pltpu.sync_copy(zerosd_hbm, accd_sh.at[pl.ds(row0, SLICE)])

    pltpu.sync_copy(edge_hbm.at[1, pl.ds(e0, EPT)], src_v)
    pltpu.sync_copy(edge_hbm.at[0, pl.ds(e0, EPT)], dst_v)

    def g_pairs(k, b):
        sl = src_v.at[pl.ds(k * CH, CH)]
        return ((g_hbm.at[sl], rows_v.at[b], gsem[b]),
                (u_hbm.at[sl], drows_v.at[b], gsem[b]))

    def s_pairs(k, b):
        sl = dst_v.at[pl.ds(k * CH, CH)]
        return ((rows_v.at[b], acc_sh.at[sl], ssem[b]),
                (drows_v.at[b], accd_sh.at[sl], ssem[b]))

    def g_issue(k, b):
        for p in g_pairs(k, b):
            pltpu.async_copy(*p)

    def g_wait(k, b):
        for p in g_pairs(k, b):
            pltpu.make_async_copy(*p).wait()

    def s_issue(k, b):
        for p in s_pairs(k, b):
            pltpu.async_copy(*p, add=True)

    def s_wait(k, b):
        for p in s_pairs(k, b):
            pltpu.make_async_copy(*p).wait()

    tl = pl.ds(KCH * CH, TAIL)

    def gt_pairs():
        return ((g_hbm.at[src_v.at[tl]], rows_v.at[0, pl.ds(0, TAIL)],
                 gsem[0]),
                (u_hbm.at[src_v.at[tl]], drows_v.at[0, pl.ds(0, TAIL)],
                 gsem[0]))

    def st_pairs():
        return ((rows_v.at[0, pl.ds(0, TAIL)], acc_sh.at[dst_v.at[tl]],
                 ssem[0]),
                (drows_v.at[0, pl.ds(0, TAIL)], accd_sh.at[dst_v.at[tl]],
                 ssem[0]))

    g_issue(0, 0)
    g_wait(0, 0)
    s_issue(0, 0)
    g_issue(1, 1)

    def body(kk, carry):
        for j in range(2):
            k = 2 * kk + 1 + j
            cur = (1 + j) % 2
            oth = 1 - cur
            s_wait(k - 1, oth)
            g_issue(k + 1, oth)
            g_wait(k, cur)
            s_issue(k, cur)
        return carry

    lax.fori_loop(0, (KCH - 2) // 2, body, 0)

    s_wait(KCH - 2, 0)
    for p in gt_pairs():
        pltpu.async_copy(*p)
    g_wait(KCH - 1, 1)
    s_issue(KCH - 1, 1)
    for p in gt_pairs():
        pltpu.make_async_copy(*p).wait()
    for p in st_pairs():
        pltpu.async_copy(*p, add=True)
    s_wait(KCH - 1, 1)
    for p in st_pairs():
        pltpu.make_async_copy(*p).wait()

    plsc.subcore_barrier()

    @pl.when(c == 0)
    def _():
        pltpu.sync_copy(acc_sh.at[pl.ds(row0, SLICE)],
                        out0_hbm.at[pl.ds(row0, SLICE)])
        pltpu.sync_copy(accd_sh.at[pl.ds(row0, SLICE)],
                        den0_hbm.at[pl.ds(row0, SLICE)])

    @pl.when(c == 1)
    def _():
        pltpu.sync_copy(acc_sh.at[pl.ds(row0, SLICE)],
                        out1_hbm.at[pl.ds(row0, SLICE)])
        pltpu.sync_copy(accd_sh.at[pl.ds(row0, SLICE)],
                        den1_hbm.at[pl.ds(row0, SLICE)])


def _sc_aggregate(g, u16, edge_index, zeros, zerosd):
    mesh = plsc.VectorSubcoreMesh(core_axis_name="c", subcore_axis_name="s")
    kern = pl.kernel(
        _sc_body,
        out_type=[jax.ShapeDtypeStruct((NPAD, D), jnp.float32),
                  jax.ShapeDtypeStruct((NPAD, D), jnp.float32),
                  jax.ShapeDtypeStruct((NPAD, DW), jnp.float32),
                  jax.ShapeDtypeStruct((NPAD, DW), jnp.float32)],
        mesh=mesh,
        scratch_types=[
            pltpu.VMEM((EPT,), jnp.int32),
            pltpu.VMEM((EPT,), jnp.int32),
            pltpu.VMEM((2, CH, D), jnp.float32),
            pltpu.VMEM((2, CH, DW), jnp.float32),
            pltpu.VMEM_SHARED((NPAD, D), jnp.float32),
            pltpu.VMEM_SHARED((NPAD, DW), jnp.float32),
            *([pltpu.SemaphoreType.DMA] * 4),
        ],
        compiler_params=pltpu.CompilerParams(use_tc_tiling_on_sc=False),
    )
    return kern(g, u16, edge_index, zeros, zerosd)


def _combine_body(s0_ref, s1_ref, d0_ref, d1_ref, h_ref, x_ref, wc_ref,
                  bc_ref, o_ref):
    den = d0_ref[:, 0:1] + d1_ref[:, 0:1]
    agg = (s0_ref[...] + s1_ref[...]) / jnp.maximum(den, 1e-16)
    wd = wc_ref[0:1, :] - wc_ref[1:2, :]
    dlt = jnp.sum(x_ref[...] * wd, axis=1, keepdims=True) + (bc_ref[0] - bc_ref[1])
    beta0 = 1.0 / (1.0 + jnp.exp(-dlt))
    out = beta0 * agg + (1.0 - beta0) * h_ref[...]
    o_ref[...] = jnp.maximum(out, 0.0)


def _combine(S0, S1, D0, D1, h, x, W_conv_pad, b_conv):
    return pl.pallas_call(
        _combine_body,
        grid=(N // BLK,),
        in_specs=[
            pl.BlockSpec((BLK, D), lambda i: (i, 0)),
            pl.BlockSpec((BLK, D), lambda i: (i, 0)),
            pl.BlockSpec((BLK, DW), lambda i: (i, 0)),
            pl.BlockSpec((BLK, DW), lambda i: (i, 0)),
            pl.BlockSpec((BLK, D), lambda i: (i, 0)),
            pl.BlockSpec((BLK, D), lambda i: (i, 0)),
            pl.BlockSpec((8, D), lambda i: (0, 0)),
            pl.BlockSpec(memory_space=pltpu.SMEM),
        ],
        out_specs=pl.BlockSpec((BLK, D), lambda i: (i, 0)),
        out_shape=jax.ShapeDtypeStruct((N, D), jnp.float32),
    )(S0, S1, D0, D1, h, x, W_conv_pad, b_conv)


def kernel(x, global_node_idx, edge_index, W_lin, b_lin, W_conv, b_conv,
           W_attn_l, b_attn_l, W_attn_r, b_attn_r, alpha_weights):
    scal = jnp.stack([b_attn_r.astype(jnp.float32).reshape(()),
                      alpha_weights.astype(jnp.float32).reshape(())])
    h, g, u16 = _prep(x, W_lin, b_lin.reshape(1, D), W_attn_r.reshape(1, D),
                      scal)

    zeros = jnp.zeros((SLICE, D), jnp.float32)
    zerosd = jnp.zeros((SLICE, DW), jnp.float32)
    S0, S1, D0, D1 = _sc_aggregate(g, u16, edge_index, zeros, zerosd)

    W_conv_pad = jnp.zeros((8, D), jnp.float32).at[:2].set(W_conv)
    return _combine(S0, S1, D0, D1, h, x, W_conv_pad, b_conv)


# speedup vs baseline: 3.5019x; 1.0127x over previous
import functools

import jax
import jax.numpy as jnp
from jax import lax
from jax.experimental import pallas as pl
from jax.experimental.pallas import tpu as pltpu
from jax.experimental.pallas import tpu_sc as plsc

N = 10000
D = 128
E = 320000
DW = 16
NT = 32
NSUB = 16
CH = 64
EPT = E // NT
KCH = EPT // CH
TAIL = EPT - KCH * CH
NPAD = N + 112
SLICE = NPAD // NSUB
BLK = 2000


def _prep_body(x_ref, wlin_ref, blin_ref, war_ref, wc_ref, scal_ref,
               h_ref, g_ref, u_ref, b_ref):
    x = x_ref[...]
    h = lax.dot_general(x, wlin_ref[...], (((1,), (1,)), ((), ())),
                        preferred_element_type=jnp.float32) + blin_ref[...]
    h_ref[...] = h
    ar = jnp.sum(h * war_ref[...], axis=1, keepdims=True) + scal_ref[0]
    u = jnp.exp(scal_ref[1] * ar)
    g_ref[...] = u * h
    u_ref[...] = jnp.broadcast_to(u, (BLK, DW))
    wd = wc_ref[0:1, :] - wc_ref[1:2, :]
    dlt = (jnp.sum(x * wd, axis=1, keepdims=True)
           + (scal_ref[2] - scal_ref[3]))
    b_ref[...] = jnp.broadcast_to(1.0 / (1.0 + jnp.exp(-dlt)), (BLK, DW))


def _prep(x, W_lin, b_lin, W_attn_r, W_conv_pad, scal):
    return pl.pallas_call(
        _prep_body,
        grid=(N // BLK,),
        in_specs=[
            pl.BlockSpec((BLK, D), lambda i: (i, 0)),
            pl.BlockSpec((D, D), lambda i: (0, 0)),
            pl.BlockSpec((1, D), lambda i: (0, 0)),
            pl.BlockSpec((1, D), lambda i: (0, 0)),
            pl.BlockSpec((8, D), lambda i: (0, 0)),
            pl.BlockSpec(memory_space=pltpu.SMEM),
        ],
        out_specs=[
            pl.BlockSpec((BLK, D), lambda i: (i, 0)),
            pl.BlockSpec((BLK, D), lambda i: (i, 0)),
            pl.BlockSpec((BLK, DW), lambda i: (i, 0)),
            pl.BlockSpec((BLK, DW), lambda i: (i, 0)),
        ],
        out_shape=[
            jax.ShapeDtypeStruct((N, D), jnp.float32),
            jax.ShapeDtypeStruct((N, D), jnp.float32),
            jax.ShapeDtypeStruct((N, DW), jnp.float32),
            jax.ShapeDtypeStruct((N, DW), jnp.float32),
        ],
    )(x, W_lin, b_lin, W_attn_r, W_conv_pad, scal)


def _sc_body(g_hbm, u_hbm, edge_hbm, zeros_hbm, zerosd_hbm,
             out0_hbm, out1_hbm, den0_hbm, den1_hbm,
             src_v, dst_v, rows_v, drows_v, acc_sh, accd_sh,
             gs0, gs1, ss0, ss1):
    gsem = (gs0, gs1)
    ssem = (ss0, ss1)
    c = lax.axis_index("c")
    s = lax.axis_index("s")
    wid = c * NSUB + s
    e0 = wid * EPT

    row0 = pl.multiple_of(s * SLICE, 8)
    pltpu.sync_copy(zeros_hbm, acc_sh.at[pl.ds(row0, SLICE)])
    pltpu.sync_copy(zerosd_hbm, accd_sh.at[pl.ds(row0, SLICE)])

    pltpu.sync_copy(edge_hbm.at[1, pl.ds(e0, EPT)], src_v)
    pltpu.sync_copy(edge_hbm.at[0, pl.ds(e0, EPT)], dst_v)

    def g_pairs(k, b):
        sl = src_v.at[pl.ds(k * CH, CH)]
        return ((g_hbm.at[sl], rows_v.at[b], gsem[b]),
                (u_hbm.at[sl], drows_v.at[b], gsem[b]))

    def s_pairs(k, b):
        sl = dst_v.at[pl.ds(k * CH, CH)]
        return ((rows_v.at[b], acc_sh.at[sl], ssem[b]),
                (drows_v.at[b], accd_sh.at[sl], ssem[b]))

    def g_issue(k, b):
        for p in g_pairs(k, b):
            pltpu.async_copy(*p)

    def g_wait(k, b):
        for p in g_pairs(k, b):
            pltpu.make_async_copy(*p).wait()

    def s_issue(k, b):
        for p in s_pairs(k, b):
            pltpu.async_copy(*p, add=True)

    def s_wait(k, b):
        for p in s_pairs(k, b):
            pltpu.make_async_copy(*p).wait()

    tl = pl.ds(KCH * CH, TAIL)

    def gt_pairs():
        return ((g_hbm.at[src_v.at[tl]], rows_v.at[0, pl.ds(0, TAIL)],
                 gsem[0]),
                (u_hbm.at[src_v.at[tl]], drows_v.at[0, pl.ds(0, TAIL)],
                 gsem[0]))

    def st_pairs():
        return ((rows_v.at[0, pl.ds(0, TAIL)], acc_sh.at[dst_v.at[tl]],
                 ssem[0]),
                (drows_v.at[0, pl.ds(0, TAIL)], accd_sh.at[dst_v.at[tl]],
                 ssem[0]))

    g_issue(0, 0)
    g_wait(0, 0)
    s_issue(0, 0)
    g_issue(1, 1)

    def body(kk, carry):
        for j in range(2):
            k = 2 * kk + 1 + j
            cur = (1 + j) % 2
            oth = 1 - cur
            s_wait(k - 1, oth)
            g_issue(k + 1, oth)
            g_wait(k, cur)
            s_issue(k, cur)
        return carry

    lax.fori_loop(0, (KCH - 2) // 2, body, 0)

    s_wait(KCH - 2, 0)
    for p in gt_pairs():
        pltpu.async_copy(*p)
    g_wait(KCH - 1, 1)
    s_issue(KCH - 1, 1)
    for p in gt_pairs():
        pltpu.make_async_copy(*p).wait()
    for p in st_pairs():
        pltpu.async_copy(*p, add=True)
    s_wait(KCH - 1, 1)
    for p in st_pairs():
        pltpu.make_async_copy(*p).wait()

    plsc.subcore_barrier()

    @pl.when(c == 0)
    def _():
        pltpu.sync_copy(acc_sh.at[pl.ds(row0, SLICE)],
                        out0_hbm.at[pl.ds(row0, SLICE)])
        pltpu.sync_copy(accd_sh.at[pl.ds(row0, SLICE)],
                        den0_hbm.at[pl.ds(row0, SLICE)])

    @pl.when(c == 1)
    def _():
        pltpu.sync_copy(acc_sh.at[pl.ds(row0, SLICE)],
                        out1_hbm.at[pl.ds(row0, SLICE)])
        pltpu.sync_copy(accd_sh.at[pl.ds(row0, SLICE)],
                        den1_hbm.at[pl.ds(row0, SLICE)])


def _sc_aggregate(g, u16, edge_index, zeros, zerosd):
    mesh = plsc.VectorSubcoreMesh(core_axis_name="c", subcore_axis_name="s")
    kern = pl.kernel(
        _sc_body,
        out_type=[jax.ShapeDtypeStruct((NPAD, D), jnp.float32),
                  jax.ShapeDtypeStruct((NPAD, D), jnp.float32),
                  jax.ShapeDtypeStruct((NPAD, DW), jnp.float32),
                  jax.ShapeDtypeStruct((NPAD, DW), jnp.float32)],
        mesh=mesh,
        scratch_types=[
            pltpu.VMEM((EPT,), jnp.int32),
            pltpu.VMEM((EPT,), jnp.int32),
            pltpu.VMEM((2, CH, D), jnp.float32),
            pltpu.VMEM((2, CH, DW), jnp.float32),
            pltpu.VMEM_SHARED((NPAD, D), jnp.float32),
            pltpu.VMEM_SHARED((NPAD, DW), jnp.float32),
            *([pltpu.SemaphoreType.DMA] * 4),
        ],
        compiler_params=pltpu.CompilerParams(use_tc_tiling_on_sc=False),
    )
    return kern(g, u16, edge_index, zeros, zerosd)


def _combine_body(s0_ref, s1_ref, d0_ref, d1_ref, h_ref, b_ref, o_ref):
    den = d0_ref[:, 0:1] + d1_ref[:, 0:1]
    agg = (s0_ref[...] + s1_ref[...]) / jnp.maximum(den, 1e-16)
    beta0 = b_ref[:, 0:1]
    out = beta0 * agg + (1.0 - beta0) * h_ref[...]
    o_ref[...] = jnp.maximum(out, 0.0)


def _combine(S0, S1, D0, D1, h, bta):
    return pl.pallas_call(
        _combine_body,
        grid=(N // BLK,),
        in_specs=[
            pl.BlockSpec((BLK, D), lambda i: (i, 0)),
            pl.BlockSpec((BLK, D), lambda i: (i, 0)),
            pl.BlockSpec((BLK, DW), lambda i: (i, 0)),
            pl.BlockSpec((BLK, DW), lambda i: (i, 0)),
            pl.BlockSpec((BLK, D), lambda i: (i, 0)),
            pl.BlockSpec((BLK, DW), lambda i: (i, 0)),
        ],
        out_specs=pl.BlockSpec((BLK, D), lambda i: (i, 0)),
        out_shape=jax.ShapeDtypeStruct((N, D), jnp.float32),
    )(S0, S1, D0, D1, h, bta)


def kernel(x, global_node_idx, edge_index, W_lin, b_lin, W_conv, b_conv,
           W_attn_l, b_attn_l, W_attn_r, b_attn_r, alpha_weights):
    scal = jnp.stack([b_attn_r.astype(jnp.float32).reshape(()),
                      alpha_weights.astype(jnp.float32).reshape(()),
                      b_conv[0], b_conv[1]])
    W_conv_pad = jnp.zeros((8, D), jnp.float32).at[:2].set(W_conv)
    h, g, u16, bta = _prep(x, W_lin, b_lin.reshape(1, D),
                           W_attn_r.reshape(1, D), W_conv_pad, scal)

    zeros = jnp.zeros((SLICE, D), jnp.float32)
    zerosd = jnp.zeros((SLICE, DW), jnp.float32)
    S0, S1, D0, D1 = _sc_aggregate(g, u16, edge_index, zeros, zerosd)

    return _combine(S0, S1, D0, D1, h, bta)


# speedup vs baseline: 3.6780x; 1.0503x over previous
import functools

import jax
import jax.numpy as jnp
from jax import lax
from jax.experimental import pallas as pl
from jax.experimental.pallas import tpu as pltpu
from jax.experimental.pallas import tpu_sc as plsc

N = 10000
D = 128
E = 320000
DW = 8
NT = 32
NSUB = 16
CH = 80
EPT = E // NT
KCH = EPT // CH
NPAD = N + 112
SLICE = NPAD // NSUB
BLK = 2000


def _prep_body(x_ref, wlin_ref, blin_ref, war_ref, wc_ref, scal_ref,
               h_ref, g_ref, u_ref, b_ref):
    x = x_ref[...]
    h = lax.dot_general(x, wlin_ref[...], (((1,), (1,)), ((), ())),
                        preferred_element_type=jnp.float32) + blin_ref[...]
    h_ref[...] = h
    ar = jnp.sum(h * war_ref[...], axis=1, keepdims=True) + scal_ref[0]
    u = jnp.exp(scal_ref[1] * ar)
    g_ref[...] = u * h
    u_ref[...] = jnp.broadcast_to(u, (BLK, DW))
    wd = wc_ref[0:1, :] - wc_ref[1:2, :]
    dlt = (jnp.sum(x * wd, axis=1, keepdims=True)
           + (scal_ref[2] - scal_ref[3]))
    b_ref[...] = jnp.broadcast_to(1.0 / (1.0 + jnp.exp(-dlt)), (BLK, DW))


def _prep(x, W_lin, b_lin, W_attn_r, W_conv_pad, scal):
    return pl.pallas_call(
        _prep_body,
        grid=(N // BLK,),
        in_specs=[
            pl.BlockSpec((BLK, D), lambda i: (i, 0)),
            pl.BlockSpec((D, D), lambda i: (0, 0)),
            pl.BlockSpec((1, D), lambda i: (0, 0)),
            pl.BlockSpec((1, D), lambda i: (0, 0)),
            pl.BlockSpec((8, D), lambda i: (0, 0)),
            pl.BlockSpec(memory_space=pltpu.SMEM),
        ],
        out_specs=[
            pl.BlockSpec((BLK, D), lambda i: (i, 0)),
            pl.BlockSpec((BLK, D), lambda i: (i, 0)),
            pl.BlockSpec((BLK, DW), lambda i: (i, 0)),
            pl.BlockSpec((BLK, DW), lambda i: (i, 0)),
        ],
        out_shape=[
            jax.ShapeDtypeStruct((N, D), jnp.float32),
            jax.ShapeDtypeStruct((N, D), jnp.float32),
            jax.ShapeDtypeStruct((N, DW), jnp.float32),
            jax.ShapeDtypeStruct((N, DW), jnp.float32),
        ],
    )(x, W_lin, b_lin, W_attn_r, W_conv_pad, scal)


def _sc_body(g_hbm, u_hbm, edge_hbm, zeros_hbm, zerosd_hbm,
             out0_hbm, out1_hbm, den0_hbm, den1_hbm,
             src_v, dst_v, rows_v, drows_v, acc_sh, accd_sh,
             gs0, gs1, ss0, ss1):
    gsem = (gs0, gs1)
    ssem = (ss0, ss1)
    c = lax.axis_index("c")
    s = lax.axis_index("s")
    wid = c * NSUB + s
    e0 = wid * EPT

    row0 = pl.multiple_of(s * SLICE, 8)
    pltpu.sync_copy(zeros_hbm, acc_sh.at[pl.ds(row0, SLICE)])
    pltpu.sync_copy(zerosd_hbm, accd_sh.at[pl.ds(row0, SLICE)])

    pltpu.sync_copy(edge_hbm.at[1, pl.ds(e0, EPT)], src_v)
    pltpu.sync_copy(edge_hbm.at[0, pl.ds(e0, EPT)], dst_v)

    def g_pairs(k, b):
        sl = src_v.at[pl.ds(k * CH, CH)]
        return ((g_hbm.at[sl], rows_v.at[b], gsem[b]),
                (u_hbm.at[sl], drows_v.at[b], gsem[b]))

    def s_pairs(k, b):
        sl = dst_v.at[pl.ds(k * CH, CH)]
        return ((rows_v.at[b], acc_sh.at[sl], ssem[b]),
                (drows_v.at[b], accd_sh.at[sl], ssem[b]))

    def g_issue(k, b):
        for p in g_pairs(k, b):
            pltpu.async_copy(*p)

    def g_wait(k, b):
        for p in g_pairs(k, b):
            pltpu.make_async_copy(*p).wait()

    def s_issue(k, b):
        for p in s_pairs(k, b):
            pltpu.async_copy(*p, add=True)

    def s_wait(k, b):
        for p in s_pairs(k, b):
            pltpu.make_async_copy(*p).wait()

    g_issue(0, 0)
    g_wait(0, 0)
    s_issue(0, 0)
    g_issue(1, 1)

    def body(kk, carry):
        for j in range(2):
            k = 2 * kk + 1 + j
            cur = (1 + j) % 2
            oth = 1 - cur
            s_wait(k - 1, oth)
            g_issue(k + 1, oth)
            g_wait(k, cur)
            s_issue(k, cur)
        return carry

    lax.fori_loop(0, (KCH - 3) // 2, body, 0)

    s_wait(KCH - 3, 0)
    g_issue(KCH - 1, 0)
    g_wait(KCH - 2, 1)
    s_issue(KCH - 2, 1)
    s_wait(KCH - 2, 1)
    g_wait(KCH - 1, 0)
    s_issue(KCH - 1, 0)
    s_wait(KCH - 1, 0)

    plsc.subcore_barrier()

    @pl.when(c == 0)
    def _():
        pltpu.sync_copy(acc_sh.at[pl.ds(row0, SLICE)],
                        out0_hbm.at[pl.ds(row0, SLICE)])
        pltpu.sync_copy(accd_sh.at[pl.ds(row0, SLICE)],
                        den0_hbm.at[pl.ds(row0, SLICE)])

    @pl.when(c == 1)
    def _():
        pltpu.sync_copy(acc_sh.at[pl.ds(row0, SLICE)],
                        out1_hbm.at[pl.ds(row0, SLICE)])
        pltpu.sync_copy(accd_sh.at[pl.ds(row0, SLICE)],
                        den1_hbm.at[pl.ds(row0, SLICE)])


def _sc_aggregate(g, u16, edge_index, zeros, zerosd):
    mesh = plsc.VectorSubcoreMesh(core_axis_name="c", subcore_axis_name="s")
    kern = pl.kernel(
        _sc_body,
        out_type=[jax.ShapeDtypeStruct((NPAD, D), jnp.float32),
                  jax.ShapeDtypeStruct((NPAD, D), jnp.float32),
                  jax.ShapeDtypeStruct((NPAD, DW), jnp.float32),
                  jax.ShapeDtypeStruct((NPAD, DW), jnp.float32)],
        mesh=mesh,
        scratch_types=[
            pltpu.VMEM((EPT,), jnp.int32),
            pltpu.VMEM((EPT,), jnp.int32),
            pltpu.VMEM((2, CH, D), jnp.float32),
            pltpu.VMEM((2, CH, DW), jnp.float32),
            pltpu.VMEM_SHARED((NPAD, D), jnp.float32),
            pltpu.VMEM_SHARED((NPAD, DW), jnp.float32),
            *([pltpu.SemaphoreType.DMA] * 4),
        ],
        compiler_params=pltpu.CompilerParams(use_tc_tiling_on_sc=False),
    )
    return kern(g, u16, edge_index, zeros, zerosd)


def _combine_body(s0_ref, s1_ref, d0_ref, d1_ref, h_ref, b_ref, o_ref):
    den = d0_ref[:, 0:1] + d1_ref[:, 0:1]
    agg = (s0_ref[...] + s1_ref[...]) / jnp.maximum(den, 1e-16)
    beta0 = b_ref[:, 0:1]
    out = beta0 * agg + (1.0 - beta0) * h_ref[...]
    o_ref[...] = jnp.maximum(out, 0.0)


def _combine(S0, S1, D0, D1, h, bta):
    return pl.pallas_call(
        _combine_body,
        grid=(N // BLK,),
        in_specs=[
            pl.BlockSpec((BLK, D), lambda i: (i, 0)),
            pl.BlockSpec((BLK, D), lambda i: (i, 0)),
            pl.BlockSpec((BLK, DW), lambda i: (i, 0)),
            pl.BlockSpec((BLK, DW), lambda i: (i, 0)),
            pl.BlockSpec((BLK, D), lambda i: (i, 0)),
            pl.BlockSpec((BLK, DW), lambda i: (i, 0)),
        ],
        out_specs=pl.BlockSpec((BLK, D), lambda i: (i, 0)),
        out_shape=jax.ShapeDtypeStruct((N, D), jnp.float32),
    )(S0, S1, D0, D1, h, bta)


def kernel(x, global_node_idx, edge_index, W_lin, b_lin, W_conv, b_conv,
           W_attn_l, b_attn_l, W_attn_r, b_attn_r, alpha_weights):
    scal = jnp.stack([b_attn_r.astype(jnp.float32).reshape(()),
                      alpha_weights.astype(jnp.float32).reshape(()),
                      b_conv[0], b_conv[1]])
    W_conv_pad = jnp.zeros((8, D), jnp.float32).at[:2].set(W_conv)
    h, g, u16, bta = _prep(x, W_lin, b_lin.reshape(1, D),
                           W_attn_r.reshape(1, D), W_conv_pad, scal)

    zeros = jnp.zeros((SLICE, D), jnp.float32)
    zerosd = jnp.zeros((SLICE, DW), jnp.float32)
    S0, S1, D0, D1 = _sc_aggregate(g, u16, edge_index, zeros, zerosd)

    return _combine(S0, S1, D0, D1, h, bta)


# speedup vs baseline: 3.8180x; 1.0381x over previous
import functools

import jax
import jax.numpy as jnp
from jax import lax
from jax.experimental import pallas as pl
from jax.experimental.pallas import tpu as pltpu
from jax.experimental.pallas import tpu_sc as plsc

N = 10000
D = 128
E = 320000
DW = 8
NT = 32
NSUB = 16
CH = 80
EPT = E // NT
KCH = EPT // CH
NPAD = N + 112
SLICE = NPAD // NSUB
BLK = 2000


def _prep_body(x_ref, wlin_ref, blin_ref, war_ref, wc_ref, scal_ref,
               h_ref, g_ref, u_ref, b_ref):
    x = x_ref[...]
    h = lax.dot_general(x, wlin_ref[...], (((1,), (1,)), ((), ())),
                        preferred_element_type=jnp.float32) + blin_ref[...]
    h_ref[...] = h
    ar = jnp.sum(h * war_ref[...], axis=1, keepdims=True) + scal_ref[0]
    u = jnp.exp(scal_ref[1] * ar)
    g_ref[...] = u * h
    u_ref[...] = jnp.broadcast_to(u, (BLK, DW))
    wd = wc_ref[0:1, :] - wc_ref[1:2, :]
    dlt = (jnp.sum(x * wd, axis=1, keepdims=True)
           + (scal_ref[2] - scal_ref[3]))
    b_ref[...] = jnp.broadcast_to(1.0 / (1.0 + jnp.exp(-dlt)), (BLK, DW))


def _prep(x, W_lin, b_lin, W_attn_r, W_conv_pad, scal):
    return pl.pallas_call(
        _prep_body,
        grid=(N // BLK,),
        in_specs=[
            pl.BlockSpec((BLK, D), lambda i: (i, 0)),
            pl.BlockSpec((D, D), lambda i: (0, 0)),
            pl.BlockSpec((1, D), lambda i: (0, 0)),
            pl.BlockSpec((1, D), lambda i: (0, 0)),
            pl.BlockSpec((8, D), lambda i: (0, 0)),
            pl.BlockSpec(memory_space=pltpu.SMEM),
        ],
        out_specs=[
            pl.BlockSpec((BLK, D), lambda i: (i, 0)),
            pl.BlockSpec((BLK, D), lambda i: (i, 0)),
            pl.BlockSpec((BLK, DW), lambda i: (i, 0)),
            pl.BlockSpec((BLK, DW), lambda i: (i, 0)),
        ],
        out_shape=[
            jax.ShapeDtypeStruct((N, D), jnp.float32),
            jax.ShapeDtypeStruct((N, D), jnp.float32),
            jax.ShapeDtypeStruct((N, DW), jnp.float32),
            jax.ShapeDtypeStruct((N, DW), jnp.float32),
        ],
    )(x, W_lin, b_lin, W_attn_r, W_conv_pad, scal)


def _sc_body(g_hbm, u_hbm, edge_hbm, zeros_hbm, zerosd_hbm,
             out0_hbm, out1_hbm, den0_hbm, den1_hbm,
             src_v, dst_v, rows_v, drows_v, acc_sh, accd_sh,
             gs0, gs1, ss0, ss1):
    gsem = (gs0, gs1)
    ssem = (ss0, ss1)
    c = lax.axis_index("c")
    s = lax.axis_index("s")
    wid = c * NSUB + s
    e0 = wid * EPT

    row0 = pl.multiple_of(s * SLICE, 8)
    pltpu.sync_copy(zeros_hbm, acc_sh.at[pl.ds(row0, SLICE)])
    pltpu.sync_copy(zerosd_hbm, accd_sh.at[pl.ds(row0, SLICE)])

    pltpu.sync_copy(edge_hbm.at[pl.ds(E + e0, EPT)], src_v)
    pltpu.sync_copy(edge_hbm.at[pl.ds(e0, EPT)], dst_v)

    def g_pairs(k, b):
        sl = src_v.at[pl.ds(k * CH, CH)]
        return ((g_hbm.at[sl], rows_v.at[b], gsem[b]),
                (u_hbm.at[sl], drows_v.at[b], gsem[b]))

    def s_pairs(k, b):
        sl = dst_v.at[pl.ds(k * CH, CH)]
        return ((rows_v.at[b], acc_sh.at[sl], ssem[b]),
                (drows_v.at[b], accd_sh.at[sl], ssem[b]))

    def g_issue(k, b):
        for p in g_pairs(k, b):
            pltpu.async_copy(*p)

    def g_wait(k, b):
        for p in g_pairs(k, b):
            pltpu.make_async_copy(*p).wait()

    def s_issue(k, b):
        for p in s_pairs(k, b):
            pltpu.async_copy(*p, add=True)

    def s_wait(k, b):
        for p in s_pairs(k, b):
            pltpu.make_async_copy(*p).wait()

    g_issue(0, 0)
    g_wait(0, 0)
    s_issue(0, 0)
    g_issue(1, 1)

    def body(kk, carry):
        for j in range(2):
            k = 2 * kk + 1 + j
            cur = (1 + j) % 2
            oth = 1 - cur
            s_wait(k - 1, oth)
            g_issue(k + 1, oth)
            g_wait(k, cur)
            s_issue(k, cur)
        return carry

    lax.fori_loop(0, (KCH - 3) // 2, body, 0)

    s_wait(KCH - 3, 0)
    g_issue(KCH - 1, 0)
    g_wait(KCH - 2, 1)
    s_issue(KCH - 2, 1)
    s_wait(KCH - 2, 1)
    g_wait(KCH - 1, 0)
    s_issue(KCH - 1, 0)
    s_wait(KCH - 1, 0)

    plsc.subcore_barrier()

    @pl.when(c == 0)
    def _():
        pltpu.sync_copy(acc_sh.at[pl.ds(row0, SLICE)],
                        out0_hbm.at[pl.ds(row0, SLICE)])
        pltpu.sync_copy(accd_sh.at[pl.ds(row0, SLICE)],
                        den0_hbm.at[pl.ds(row0, SLICE), pl.ds(0, DW)])

    @pl.when(c == 1)
    def _():
        pltpu.sync_copy(acc_sh.at[pl.ds(row0, SLICE)],
                        out1_hbm.at[pl.ds(row0, SLICE)])
        pltpu.sync_copy(accd_sh.at[pl.ds(row0, SLICE)],
                        den1_hbm.at[pl.ds(row0, SLICE), pl.ds(0, DW)])


def _sc_aggregate(g, u16, edge_index, zeros, zerosd):
    mesh = plsc.VectorSubcoreMesh(core_axis_name="c", subcore_axis_name="s")
    kern = pl.kernel(
        _sc_body,
        out_type=[jax.ShapeDtypeStruct((NPAD, D), jnp.float32),
                  jax.ShapeDtypeStruct((NPAD, D), jnp.float32),
                  jax.ShapeDtypeStruct((NPAD, D), jnp.float32),
                  jax.ShapeDtypeStruct((NPAD, D), jnp.float32)],
        mesh=mesh,
        scratch_types=[
            pltpu.VMEM((EPT,), jnp.int32),
            pltpu.VMEM((EPT,), jnp.int32),
            pltpu.VMEM((2, CH, D), jnp.float32),
            pltpu.VMEM((2, CH, DW), jnp.float32),
            pltpu.VMEM_SHARED((NPAD, D), jnp.float32),
            pltpu.VMEM_SHARED((NPAD, DW), jnp.float32),
            *([pltpu.SemaphoreType.DMA] * 4),
        ],
        compiler_params=pltpu.CompilerParams(use_tc_tiling_on_sc=False),
    )
    return kern(g, u16, edge_index, zeros, zerosd)


def _combine_body(s0_ref, s1_ref, d0_ref, d1_ref, h_ref, b_ref, o_ref):
    den = d0_ref[:, 0:1] + d1_ref[:, 0:1]
    agg = (s0_ref[...] + s1_ref[...]) / jnp.maximum(den, 1e-16)
    beta0 = b_ref[:, 0:1]
    out = beta0 * agg + (1.0 - beta0) * h_ref[...]
    o_ref[...] = jnp.maximum(out, 0.0)


def _combine(S0, S1, D0, D1, h, bta):
    return pl.pallas_call(
        _combine_body,
        grid=(N // BLK,),
        in_specs=[
            pl.BlockSpec((BLK, D), lambda i: (i, 0)),
            pl.BlockSpec((BLK, D), lambda i: (i, 0)),
            pl.BlockSpec((BLK, D), lambda i: (i, 0)),
            pl.BlockSpec((BLK, D), lambda i: (i, 0)),
            pl.BlockSpec((BLK, D), lambda i: (i, 0)),
            pl.BlockSpec((BLK, DW), lambda i: (i, 0)),
        ],
        out_specs=pl.BlockSpec((BLK, D), lambda i: (i, 0)),
        out_shape=jax.ShapeDtypeStruct((N, D), jnp.float32),
    )(S0, S1, D0, D1, h, bta)


def kernel(x, global_node_idx, edge_index, W_lin, b_lin, W_conv, b_conv,
           W_attn_l, b_attn_l, W_attn_r, b_attn_r, alpha_weights):
    scal = jnp.stack([b_attn_r.astype(jnp.float32).reshape(()),
                      alpha_weights.astype(jnp.float32).reshape(()),
                      b_conv[0], b_conv[1]])
    W_conv_pad = jnp.zeros((8, D), jnp.float32).at[:2].set(W_conv)
    h, g, u16, bta = _prep(x, W_lin, b_lin.reshape(1, D),
                           W_attn_r.reshape(1, D), W_conv_pad, scal)

    zeros = jnp.zeros((SLICE, D), jnp.float32)
    zerosd = jnp.zeros((SLICE, DW), jnp.float32)
    S0, S1, D0, D1 = _sc_aggregate(g, u16, edge_index.reshape(2 * E),
                                   zeros, zerosd)

    return _combine(S0, S1, D0, D1, h, bta)


# speedup vs baseline: 3.8202x; 1.0006x over previous
import functools

import jax
import jax.numpy as jnp
from jax import lax
from jax.experimental import pallas as pl
from jax.experimental.pallas import tpu as pltpu
from jax.experimental.pallas import tpu_sc as plsc

N = 10000
D = 128
E = 320000
DW = 8
NT = 32
NSUB = 16
CH = 80
EPT = E // NT
KCH = EPT // CH
NPAD = N + 112
SLICE = NPAD // NSUB
BLK = 2000


def _prep_body(x_ref, wlin_ref, blin_ref, war_ref, wc_ref, scal_ref,
               h_ref, g_ref, u_ref, b_ref):
    x = x_ref[...]
    h = lax.dot_general(x, wlin_ref[...], (((1,), (1,)), ((), ())),
                        preferred_element_type=jnp.float32) + blin_ref[...]
    h_ref[...] = h
    ar = jnp.sum(h * war_ref[...], axis=1, keepdims=True) + scal_ref[0]
    u = jnp.exp(scal_ref[1] * ar)
    g_ref[...] = u * h
    u_ref[...] = jnp.broadcast_to(u, (BLK, DW))
    wd = wc_ref[0:1, :] - wc_ref[1:2, :]
    dlt = (jnp.sum(x * wd, axis=1, keepdims=True)
           + (scal_ref[2] - scal_ref[3]))
    b_ref[...] = jnp.broadcast_to(1.0 / (1.0 + jnp.exp(-dlt)), (BLK, D))


def _prep(x, W_lin, b_lin, W_attn_r, W_conv_pad, scal):
    return pl.pallas_call(
        _prep_body,
        grid=(N // BLK,),
        in_specs=[
            pl.BlockSpec((BLK, D), lambda i: (i, 0)),
            pl.BlockSpec((D, D), lambda i: (0, 0)),
            pl.BlockSpec((1, D), lambda i: (0, 0)),
            pl.BlockSpec((1, D), lambda i: (0, 0)),
            pl.BlockSpec((8, D), lambda i: (0, 0)),
            pl.BlockSpec(memory_space=pltpu.SMEM),
        ],
        out_specs=[
            pl.BlockSpec((BLK, D), lambda i: (i, 0)),
            pl.BlockSpec((BLK, D), lambda i: (i, 0)),
            pl.BlockSpec((BLK, DW), lambda i: (i, 0)),
            pl.BlockSpec((BLK, D), lambda i: (i, 0)),
        ],
        out_shape=[
            jax.ShapeDtypeStruct((N, D), jnp.float32),
            jax.ShapeDtypeStruct((N, D), jnp.float32),
            jax.ShapeDtypeStruct((N, DW), jnp.float32),
            jax.ShapeDtypeStruct((N, D), jnp.float32),
        ],
    )(x, W_lin, b_lin, W_attn_r, W_conv_pad, scal)


def _sc_body(g_hbm, u_hbm, edge_hbm, zeros_hbm, zerosd_hbm,
             out0_hbm, out1_hbm, den0_hbm, den1_hbm,
             src_v, dst_v, rows_v, drows_v, acc_sh, accd_sh,
             gs0, gs1, ss0, ss1):
    gsem = (gs0, gs1)
    ssem = (ss0, ss1)
    c = lax.axis_index("c")
    s = lax.axis_index("s")
    wid = c * NSUB + s
    e0 = wid * EPT

    row0 = pl.multiple_of(s * SLICE, 8)
    pltpu.sync_copy(zeros_hbm, acc_sh.at[pl.ds(row0, SLICE)])
    pltpu.sync_copy(zerosd_hbm, accd_sh.at[pl.ds(row0, SLICE)])

    pltpu.sync_copy(edge_hbm.at[pl.ds(E + e0, EPT)], src_v)
    pltpu.sync_copy(edge_hbm.at[pl.ds(e0, EPT)], dst_v)

    def g_pairs(k, b):
        sl = src_v.at[pl.ds(k * CH, CH)]
        return ((g_hbm.at[sl], rows_v.at[b], gsem[b]),
                (u_hbm.at[sl], drows_v.at[b], gsem[b]))

    def s_pairs(k, b):
        sl = dst_v.at[pl.ds(k * CH, CH)]
        return ((rows_v.at[b], acc_sh.at[sl], ssem[b]),
                (drows_v.at[b], accd_sh.at[sl], ssem[b]))

    def g_issue(k, b):
        for p in g_pairs(k, b):
            pltpu.async_copy(*p)

    def g_wait(k, b):
        for p in g_pairs(k, b):
            pltpu.make_async_copy(*p).wait()

    def s_issue(k, b):
        for p in s_pairs(k, b):
            pltpu.async_copy(*p, add=True)

    def s_wait(k, b):
        for p in s_pairs(k, b):
            pltpu.make_async_copy(*p).wait()

    g_issue(0, 0)
    g_wait(0, 0)
    s_issue(0, 0)
    g_issue(1, 1)

    def body(kk, carry):
        for j in range(2):
            k = 2 * kk + 1 + j
            cur = (1 + j) % 2
            oth = 1 - cur
            s_wait(k - 1, oth)
            g_issue(k + 1, oth)
            g_wait(k, cur)
            s_issue(k, cur)
        return carry

    lax.fori_loop(0, (KCH - 3) // 2, body, 0)

    s_wait(KCH - 3, 0)
    g_issue(KCH - 1, 0)
    g_wait(KCH - 2, 1)
    s_issue(KCH - 2, 1)
    s_wait(KCH - 2, 1)
    g_wait(KCH - 1, 0)
    s_issue(KCH - 1, 0)
    s_wait(KCH - 1, 0)

    plsc.subcore_barrier()

    @pl.when(c == 0)
    def _():
        pltpu.sync_copy(acc_sh.at[pl.ds(row0, SLICE)],
                        out0_hbm.at[pl.ds(row0, SLICE)])
        pltpu.sync_copy(accd_sh.at[pl.ds(row0, SLICE)],
                        den0_hbm.at[pl.ds(row0, SLICE), pl.ds(0, DW)])

    @pl.when(c == 1)
    def _():
        pltpu.sync_copy(acc_sh.at[pl.ds(row0, SLICE)],
                        out1_hbm.at[pl.ds(row0, SLICE)])
        pltpu.sync_copy(accd_sh.at[pl.ds(row0, SLICE)],
                        den1_hbm.at[pl.ds(row0, SLICE), pl.ds(0, DW)])


def _sc_aggregate(g, u16, edge_index, zeros, zerosd):
    mesh = plsc.VectorSubcoreMesh(core_axis_name="c", subcore_axis_name="s")
    kern = pl.kernel(
        _sc_body,
        out_type=[jax.ShapeDtypeStruct((NPAD, D), jnp.float32),
                  jax.ShapeDtypeStruct((NPAD, D), jnp.float32),
                  jax.ShapeDtypeStruct((NPAD, D), jnp.float32),
                  jax.ShapeDtypeStruct((NPAD, D), jnp.float32)],
        mesh=mesh,
        scratch_types=[
            pltpu.VMEM((EPT,), jnp.int32),
            pltpu.VMEM((EPT,), jnp.int32),
            pltpu.VMEM((2, CH, D), jnp.float32),
            pltpu.VMEM((2, CH, DW), jnp.float32),
            pltpu.VMEM_SHARED((NPAD, D), jnp.float32),
            pltpu.VMEM_SHARED((NPAD, DW), jnp.float32),
            *([pltpu.SemaphoreType.DMA] * 4),
        ],
        compiler_params=pltpu.CompilerParams(use_tc_tiling_on_sc=False),
    )
    return kern(g, u16, edge_index, zeros, zerosd)


def _combine_body(s0_ref, s1_ref, d0_ref, d1_ref, h_ref, b_ref, o_ref):
    den = d0_ref[:, 0:1] + d1_ref[:, 0:1]
    agg = (s0_ref[...] + s1_ref[...]) / jnp.maximum(den, 1e-16)
    beta0 = b_ref[:, 0:1]
    out = beta0 * agg + (1.0 - beta0) * h_ref[...]
    o_ref[...] = jnp.maximum(out, 0.0)


def _combine(S0, S1, D0, D1, h, bta):
    return pl.pallas_call(
        _combine_body,
        grid=(N // BLK,),
        in_specs=[
            pl.BlockSpec((BLK, D), lambda i: (i, 0)),
            pl.BlockSpec((BLK, D), lambda i: (i, 0)),
            pl.BlockSpec((BLK, D), lambda i: (i, 0)),
            pl.BlockSpec((BLK, D), lambda i: (i, 0)),
            pl.BlockSpec((BLK, D), lambda i: (i, 0)),
            pl.BlockSpec((BLK, D), lambda i: (i, 0)),
        ],
        out_specs=pl.BlockSpec((BLK, D), lambda i: (i, 0)),
        out_shape=jax.ShapeDtypeStruct((N, D), jnp.float32),
    )(S0, S1, D0, D1, h, bta)


def kernel(x, global_node_idx, edge_index, W_lin, b_lin, W_conv, b_conv,
           W_attn_l, b_attn_l, W_attn_r, b_attn_r, alpha_weights):
    scal = jnp.stack([b_attn_r.astype(jnp.float32).reshape(()),
                      alpha_weights.astype(jnp.float32).reshape(()),
                      b_conv[0], b_conv[1]])
    W_conv_pad = jnp.zeros((8, D), jnp.float32).at[:2].set(W_conv)
    h, g, u16, bta = _prep(x, W_lin, b_lin.reshape(1, D),
                           W_attn_r.reshape(1, D), W_conv_pad, scal)

    zeros = jnp.zeros((SLICE, D), jnp.float32)
    zerosd = jnp.zeros((SLICE, DW), jnp.float32)
    S0, S1, D0, D1 = _sc_aggregate(g, u16, edge_index.reshape(2 * E),
                                   zeros, zerosd)

    return _combine(S0, S1, D0, D1, h, bta)


# speedup vs baseline: 3.8207x; 1.0001x over previous
import jax
import jax.numpy as jnp
from jax import lax
from jax.experimental import pallas as pl
from jax.experimental.pallas import tpu as pltpu
from jax.experimental.pallas import tpu_sc as plsc

N = 10000
D = 128
E = 320000
DW = 8
NT = 32
NSUB = 16
CH = 80
EPT = E // NT
KCH = EPT // CH
NPAD = N + 112
SLICE = NPAD // NSUB
BLK = 2000


def _prep_body(x_ref, wlin_ref, blin_ref, war_ref, wc_ref, scal_ref,
               h_ref, g_ref, u_ref, b_ref):
    x = x_ref[...]
    h = lax.dot_general(x, wlin_ref[...], (((1,), (1,)), ((), ())),
                        preferred_element_type=jnp.float32) + blin_ref[...]
    h_ref[...] = h
    ar = jnp.sum(h * war_ref[...], axis=1, keepdims=True) + scal_ref[0]
    u = jnp.exp(scal_ref[1] * ar)
    g_ref[...] = u * h
    u_ref[...] = jnp.broadcast_to(u, (BLK, DW))
    wd = wc_ref[0:1, :] - wc_ref[1:2, :]
    dlt = (jnp.sum(x * wd, axis=1, keepdims=True)
           + (scal_ref[2] - scal_ref[3]))
    b_ref[...] = jnp.broadcast_to(1.0 / (1.0 + jnp.exp(-dlt)), (BLK, D))


def _prep(x, W_lin, b_lin, W_attn_r, W_conv_pad, scal):
    return pl.pallas_call(
        _prep_body,
        grid=(N // BLK,),
        in_specs=[
            pl.BlockSpec((BLK, D), lambda i: (i, 0)),
            pl.BlockSpec((D, D), lambda i: (0, 0)),
            pl.BlockSpec((1, D), lambda i: (0, 0)),
            pl.BlockSpec((1, D), lambda i: (0, 0)),
            pl.BlockSpec((8, D), lambda i: (0, 0)),
            pl.BlockSpec(memory_space=pltpu.SMEM),
        ],
        out_specs=[
            pl.BlockSpec((BLK, D), lambda i: (i, 0)),
            pl.BlockSpec((BLK, D), lambda i: (i, 0)),
            pl.BlockSpec((BLK, DW), lambda i: (i, 0)),
            pl.BlockSpec((BLK, D), lambda i: (i, 0)),
        ],
        out_shape=[
            jax.ShapeDtypeStruct((N, D), jnp.float32),
            jax.ShapeDtypeStruct((N, D), jnp.float32),
            jax.ShapeDtypeStruct((N, DW), jnp.float32),
            jax.ShapeDtypeStruct((N, D), jnp.float32),
        ],
    )(x, W_lin, b_lin, W_attn_r, W_conv_pad, scal)


def _sc_body(g_hbm, u_hbm, edge_hbm, zeros_hbm, zerosd_hbm,
             out0_hbm, out1_hbm, den0_hbm, den1_hbm,
             src_v, dst_v, rows_v, drows_v, acc_sh, accd_sh,
             gs0, gs1, ss0, ss1):
    gsem = (gs0, gs1)
    ssem = (ss0, ss1)
    c = lax.axis_index("c")
    s = lax.axis_index("s")
    wid = c * NSUB + s
    e0 = wid * EPT

    row0 = pl.multiple_of(s * SLICE, 8)
    pltpu.sync_copy(zeros_hbm, acc_sh.at[pl.ds(row0, SLICE)])
    pltpu.sync_copy(zerosd_hbm, accd_sh.at[pl.ds(row0, SLICE)])

    pltpu.sync_copy(edge_hbm.at[pl.ds(E + e0, EPT)], src_v)
    pltpu.sync_copy(edge_hbm.at[pl.ds(e0, EPT)], dst_v)

    def g_pairs(k, b):
        sl = src_v.at[pl.ds(k * CH, CH)]
        return ((g_hbm.at[sl], rows_v.at[b], gsem[b]),
                (u_hbm.at[sl], drows_v.at[b], gsem[b]))

    def s_pairs(k, b):
        sl = dst_v.at[pl.ds(k * CH, CH)]
        return ((rows_v.at[b], acc_sh.at[sl], ssem[b]),
                (drows_v.at[b], accd_sh.at[sl], ssem[b]))

    def g_issue(k, b):
        for p in g_pairs(k, b):
            pltpu.async_copy(*p)

    def g_wait(k, b):
        for p in g_pairs(k, b):
            pltpu.make_async_copy(*p).wait()

    def s_issue(k, b):
        for p in s_pairs(k, b):
            pltpu.async_copy(*p, add=True)

    def s_wait(k, b):
        for p in s_pairs(k, b):
            pltpu.make_async_copy(*p).wait()

    g_issue(0, 0)
    g_wait(0, 0)
    s_issue(0, 0)
    g_issue(1, 1)

    def body(kk, carry):
        for j in range(2):
            k = 2 * kk + 1 + j
            cur = (1 + j) % 2
            oth = 1 - cur
            s_wait(k - 1, oth)
            g_issue(k + 1, oth)
            g_wait(k, cur)
            s_issue(k, cur)
        return carry

    lax.fori_loop(0, (KCH - 3) // 2, body, 0)

    s_wait(KCH - 3, 0)
    g_issue(KCH - 1, 0)
    g_wait(KCH - 2, 1)
    s_issue(KCH - 2, 1)
    s_wait(KCH - 2, 1)
    g_wait(KCH - 1, 0)
    s_issue(KCH - 1, 0)
    s_wait(KCH - 1, 0)

    plsc.subcore_barrier()

    @pl.when(c == 0)
    def _():
        pltpu.sync_copy(acc_sh.at[pl.ds(row0, SLICE)],
                        out0_hbm.at[pl.ds(row0, SLICE)])
        pltpu.sync_copy(accd_sh.at[pl.ds(row0, SLICE)],
                        den0_hbm.at[pl.ds(row0, SLICE), pl.ds(0, DW)])

    @pl.when(c == 1)
    def _():
        pltpu.sync_copy(acc_sh.at[pl.ds(row0, SLICE)],
                        out1_hbm.at[pl.ds(row0, SLICE)])
        pltpu.sync_copy(accd_sh.at[pl.ds(row0, SLICE)],
                        den1_hbm.at[pl.ds(row0, SLICE), pl.ds(0, DW)])


def _sc_aggregate(g, u16, edge_index, zeros, zerosd):
    mesh = plsc.VectorSubcoreMesh(core_axis_name="c", subcore_axis_name="s")
    kern = pl.kernel(
        _sc_body,
        out_type=[jax.ShapeDtypeStruct((NPAD, D), jnp.float32),
                  jax.ShapeDtypeStruct((NPAD, D), jnp.float32),
                  jax.ShapeDtypeStruct((NPAD, D), jnp.float32),
                  jax.ShapeDtypeStruct((NPAD, D), jnp.float32)],
        mesh=mesh,
        scratch_types=[
            pltpu.VMEM((EPT,), jnp.int32),
            pltpu.VMEM((EPT,), jnp.int32),
            pltpu.VMEM((2, CH, D), jnp.float32),
            pltpu.VMEM((2, CH, DW), jnp.float32),
            pltpu.VMEM_SHARED((NPAD, D), jnp.float32),
            pltpu.VMEM_SHARED((NPAD, DW), jnp.float32),
            *([pltpu.SemaphoreType.DMA] * 4),
        ],
        compiler_params=pltpu.CompilerParams(use_tc_tiling_on_sc=False),
    )
    return kern(g, u16, edge_index, zeros, zerosd)


def _combine_body(s0_ref, s1_ref, d0_ref, d1_ref, h_ref, b_ref, o_ref):
    den = d0_ref[:, 0:1] + d1_ref[:, 0:1]
    agg = (s0_ref[...] + s1_ref[...]) / jnp.maximum(den, 1e-16)
    beta0 = b_ref[:, 0:1]
    out = beta0 * agg + (1.0 - beta0) * h_ref[...]
    o_ref[...] = jnp.maximum(out, 0.0)


def _combine(S0, S1, D0, D1, h, bta):
    return pl.pallas_call(
        _combine_body,
        grid=(N // BLK,),
        in_specs=[
            pl.BlockSpec((BLK, D), lambda i: (i, 0)),
            pl.BlockSpec((BLK, D), lambda i: (i, 0)),
            pl.BlockSpec((BLK, D), lambda i: (i, 0)),
            pl.BlockSpec((BLK, D), lambda i: (i, 0)),
            pl.BlockSpec((BLK, D), lambda i: (i, 0)),
            pl.BlockSpec((BLK, D), lambda i: (i, 0)),
        ],
        out_specs=pl.BlockSpec((BLK, D), lambda i: (i, 0)),
        out_shape=jax.ShapeDtypeStruct((N, D), jnp.float32),
    )(S0, S1, D0, D1, h, bta)


def kernel(x, global_node_idx, edge_index, W_lin, b_lin, W_conv, b_conv,
           W_attn_l, b_attn_l, W_attn_r, b_attn_r, alpha_weights):
    scal = jnp.stack([b_attn_r.astype(jnp.float32).reshape(()),
                      alpha_weights.astype(jnp.float32).reshape(()),
                      b_conv[0], b_conv[1]])
    W_conv_pad = jnp.zeros((8, D), jnp.float32).at[:2].set(W_conv)
    h, g, u16, bta = _prep(x, W_lin, b_lin.reshape(1, D),
                           W_attn_r.reshape(1, D), W_conv_pad, scal)

    zeros = jnp.zeros((SLICE, D), jnp.float32)
    zerosd = jnp.zeros((SLICE, DW), jnp.float32)
    S0, S1, D0, D1 = _sc_aggregate(g, u16, edge_index.reshape(2 * E),
                                   zeros, zerosd)

    return _combine(S0, S1, D0, D1, h, bta)
